# Initial kernel scaffold; baseline (speedup 1.0000x reference)
import jax
import jax.numpy as jnp
from jax.experimental import pallas as pl


def kernel(node_repesentations, edges, edge_weights, Wp1, bp1, Wp2, bp2, Wu1, bu1, Wu2, bu2):
    raise NotImplementedError("write your pallas kernel here")


# speedup vs baseline: 7.0712x; 7.0712x over previous
import dataclasses
import functools

import jax
import jax.numpy as jnp
from jax import lax
from jax.experimental import pallas as pl
from jax.experimental.pallas import tpu as pltpu
from jax.experimental.pallas import tpu_sc as plsc

_NC = 2
_NS = 16
_LANES = 16


def _gelu(x):
    return 0.5 * x * (1.0 + lax.erf(x * 0.7071067811865476))


def _prepare_body(x_ref, w1_ref, b1_ref, w2_ref, b2_ref, o_ref):
    h = _gelu(
        jnp.dot(x_ref[...], w1_ref[...], preferred_element_type=jnp.float32)
        + b1_ref[...])
    o_ref[...] = _gelu(
        jnp.dot(h, w2_ref[...], preferred_element_type=jnp.float32) + b2_ref[...])


def _prepare_ffn(x, w1, b1, w2, b2, block_n):
    n, d = x.shape
    h = w1.shape[1]
    grid = n // block_n
    return pl.pallas_call(
        _prepare_body,
        grid=(grid,),
        in_specs=[
            pl.BlockSpec((block_n, d), lambda i: (i, 0)),
            pl.BlockSpec((d, h), lambda i: (0, 0)),
            pl.BlockSpec((1, h), lambda i: (0, 0)),
            pl.BlockSpec((h, h), lambda i: (0, 0)),
            pl.BlockSpec((1, h), lambda i: (0, 0)),
        ],
        out_specs=pl.BlockSpec((block_n, h), lambda i: (i, 0)),
        out_shape=jax.ShapeDtypeStruct((n, h), jnp.float32),
    )(x, w1, b1, w2, b2)


def _update_body(x_ref, p0_ref, p1_ref, w1x_ref, w1a_ref, b1_ref, w2_ref,
                 b2_ref, o_ref):
    agg = p0_ref[...] + p1_ref[...]
    h = _gelu(
        jnp.dot(x_ref[...], w1x_ref[...], preferred_element_type=jnp.float32)
        + jnp.dot(agg, w1a_ref[...], preferred_element_type=jnp.float32)
        + b1_ref[...])
    o = _gelu(
        jnp.dot(h, w2_ref[...], preferred_element_type=jnp.float32) + b2_ref[...])
    sq = jnp.sum(o * o, axis=-1, keepdims=True)
    o_ref[...] = o * lax.rsqrt(jnp.maximum(sq, 1e-12))


def _update_ffn(x, p0, p1, w1x, w1a, b1, w2, b2, block_n):
    n, d = x.shape
    h = w2.shape[0]
    grid = n // block_n
    return pl.pallas_call(
        _update_body,
        grid=(grid,),
        in_specs=[
            pl.BlockSpec((block_n, d), lambda i: (i, 0)),
            pl.BlockSpec((block_n, h), lambda i: (i, 0)),
            pl.BlockSpec((block_n, h), lambda i: (i, 0)),
            pl.BlockSpec((d, h), lambda i: (0, 0)),
            pl.BlockSpec((h, h), lambda i: (0, 0)),
            pl.BlockSpec((1, h), lambda i: (0, 0)),
            pl.BlockSpec((h, h), lambda i: (0, 0)),
            pl.BlockSpec((1, h), lambda i: (0, 0)),
        ],
        out_specs=pl.BlockSpec((block_n, h), lambda i: (i, 0)),
        out_shape=jax.ShapeDtypeStruct((n, h), jnp.float32),
    )(x, p0, p1, w1x, w1a, b1, w2, b2)


def _sc_spmm(h2, src, dst, w, n_nodes, window):
    e = src.shape[0]
    hdim = h2.shape[1]
    nw = _NC * _NS
    ept = e // nw
    nwin = ept // window
    rps = -(-n_nodes // _NS)
    rps = -(-rps // 8) * 8
    n_pad = rps * _NS

    src3 = src.reshape(nw, nwin, window)
    dst3 = dst.reshape(nw, nwin, window)
    w2d = w.reshape(nw, ept)
    zeros = jnp.zeros((n_pad, hdim), jnp.float32)

    mesh = plsc.VectorSubcoreMesh(core_axis_name="c", subcore_axis_name="s")
    cp = pltpu.CompilerParams()
    for fld, val in (("needs_layout_passes", False),
                     ("use_tc_tiling_on_sc", False)):
        if fld in pltpu.CompilerParams.__dataclass_fields__:
            cp = dataclasses.replace(cp, **{fld: val})

    @functools.partial(
        pl.kernel,
        mesh=mesh,
        compiler_params=cp,
        out_type=jax.ShapeDtypeStruct((_NC, n_pad, hdim), jnp.float32),
        scratch_types=[
            pltpu.VMEM((nwin, window), jnp.int32),
            pltpu.VMEM((nwin, window), jnp.int32),
            pltpu.VMEM((ept,), jnp.float32),
            pltpu.VMEM((window, hdim), jnp.float32),
            pltpu.VMEM_SHARED((n_pad, hdim), jnp.float32),
            pltpu.SemaphoreType.DMA,
        ],
    )
    def k(h2_hbm, src_hbm, dst_hbm, w_hbm, z_hbm, out_hbm,
          src_v, dst_v, w_v, rows_v, acc, sem):
        cid = lax.axis_index("c")
        sid = lax.axis_index("s")
        wid = sid * _NC + cid

        pltpu.sync_copy(src_hbm.at[wid], src_v)
        pltpu.sync_copy(dst_hbm.at[wid], dst_v)
        pltpu.sync_copy(w_hbm.at[wid], w_v)
        pltpu.sync_copy(z_hbm.at[pl.ds(sid * rps, rps)],
                        acc.at[pl.ds(sid * rps, rps)])
        plsc.subcore_barrier()

        @pl.loop(0, nwin)
        def _win(g):
            pltpu.async_copy(h2_hbm.at[src_v.at[g]], rows_v, sem).wait()

            @pl.loop(0, window)
            def _row(i):
                wv = plsc.load_gather(
                    w_v, [jnp.full((_LANES,), g * window + i, jnp.int32)])
                for c in range(hdim // _LANES):
                    sl = (i, pl.ds(c * _LANES, _LANES))
                    rows_v[sl] = rows_v[sl] * wv

            pltpu.sync_copy(rows_v, acc.at[dst_v.at[g]], add=True)

        plsc.subcore_barrier()
        pltpu.sync_copy(acc.at[pl.ds(sid * rps, rps)],
                        out_hbm.at[cid, pl.ds(sid * rps, rps)])

    return k(h2, src3, dst3, w2d, zeros)


def kernel(node_repesentations, edges, edge_weights,
           Wp1, bp1, Wp2, bp2, Wu1, bu1, Wu2, bu2):
    x = node_repesentations
    n, d = x.shape
    hdim = Wp1.shape[1]

    block_n = 1000 if n % 1000 == 0 else 8
    h2 = _prepare_ffn(x, Wp1, bp1.reshape(1, hdim), Wp2, bp2.reshape(1, hdim),
                      block_n)

    ept = edges.shape[1] // (_NC * _NS)
    window = 80 if ept % 80 == 0 else _LANES
    partials = _sc_spmm(h2, edges[1], edges[0], edge_weights, n, window)

    out = _update_ffn(x, partials[0, :n], partials[1, :n],
                      Wu1[:d], Wu1[d:], bu1.reshape(1, hdim),
                      Wu2, bu2.reshape(1, hdim), block_n)
    return out


# speedup vs baseline: 9.9543x; 1.4077x over previous
import dataclasses
import functools

import jax
import jax.numpy as jnp
from jax import lax
from jax.experimental import pallas as pl
from jax.experimental.pallas import tpu as pltpu
from jax.experimental.pallas import tpu_sc as plsc

_NC = 2
_NS = 16
_LANES = 16


def _gelu(x):
    return 0.5 * x * (1.0 + lax.erf(x * 0.7071067811865476))


def _prepare_body(x_ref, w1_ref, b1_ref, w2_ref, b2_ref, o_ref):
    h = _gelu(
        jnp.dot(x_ref[...], w1_ref[...], preferred_element_type=jnp.float32)
        + b1_ref[...])
    o_ref[...] = _gelu(
        jnp.dot(h, w2_ref[...], preferred_element_type=jnp.float32) + b2_ref[...])


def _prepare_ffn(x, w1, b1, w2, b2, block_n):
    n, d = x.shape
    h = w1.shape[1]
    grid = n // block_n
    return pl.pallas_call(
        _prepare_body,
        grid=(grid,),
        in_specs=[
            pl.BlockSpec((block_n, d), lambda i: (i, 0)),
            pl.BlockSpec((d, h), lambda i: (0, 0)),
            pl.BlockSpec((1, h), lambda i: (0, 0)),
            pl.BlockSpec((h, h), lambda i: (0, 0)),
            pl.BlockSpec((1, h), lambda i: (0, 0)),
        ],
        out_specs=pl.BlockSpec((block_n, h), lambda i: (i, 0)),
        out_shape=jax.ShapeDtypeStruct((n, h), jnp.float32),
    )(x, w1, b1, w2, b2)


def _update_body(x_ref, p0_ref, p1_ref, w1x_ref, w1a_ref, b1_ref, w2_ref,
                 b2_ref, o_ref):
    agg = p0_ref[0] + p1_ref[0]
    h = _gelu(
        jnp.dot(x_ref[...], w1x_ref[...], preferred_element_type=jnp.float32)
        + jnp.dot(agg, w1a_ref[...], preferred_element_type=jnp.float32)
        + b1_ref[...])
    o = _gelu(
        jnp.dot(h, w2_ref[...], preferred_element_type=jnp.float32) + b2_ref[...])
    sq = jnp.sum(o * o, axis=-1, keepdims=True)
    o_ref[...] = o * lax.rsqrt(jnp.maximum(sq, 1e-12))


def _update_ffn(x, partials, w1x, w1a, b1, w2, b2, block_n):
    n, d = x.shape
    h = w2.shape[0]
    grid = n // block_n
    return pl.pallas_call(
        _update_body,
        grid=(grid,),
        in_specs=[
            pl.BlockSpec((block_n, d), lambda i: (i, 0)),
            pl.BlockSpec((1, block_n, h), lambda i: (0, i, 0)),
            pl.BlockSpec((1, block_n, h), lambda i: (1, i, 0)),
            pl.BlockSpec((d, h), lambda i: (0, 0)),
            pl.BlockSpec((h, h), lambda i: (0, 0)),
            pl.BlockSpec((1, h), lambda i: (0, 0)),
            pl.BlockSpec((h, h), lambda i: (0, 0)),
            pl.BlockSpec((1, h), lambda i: (0, 0)),
        ],
        out_specs=pl.BlockSpec((block_n, h), lambda i: (i, 0)),
        out_shape=jax.ShapeDtypeStruct((n, h), jnp.float32),
    )(x, partials, partials, w1x, w1a, b1, w2, b2)


def _sc_spmm(h2, src, dst, w, n_nodes, window):
    e = src.shape[0]
    hdim = h2.shape[1]
    nw = _NC * _NS
    ept = e // nw
    nwin = ept // window
    rps = -(-n_nodes // _NS)
    rps = -(-rps // 8) * 8
    n_pad = rps * _NS

    src3 = src.reshape(nw, nwin, window)
    dst3 = dst.reshape(nw, nwin, window)
    w2d = w.reshape(nw, ept)

    mesh = plsc.VectorSubcoreMesh(core_axis_name="c", subcore_axis_name="s")
    cp = pltpu.CompilerParams()
    for fld, val in (("needs_layout_passes", False),
                     ("use_tc_tiling_on_sc", False)):
        if fld in pltpu.CompilerParams.__dataclass_fields__:
            cp = dataclasses.replace(cp, **{fld: val})

    @functools.partial(
        pl.kernel,
        mesh=mesh,
        compiler_params=cp,
        out_type=jax.ShapeDtypeStruct((_NC, n_pad, hdim), jnp.float32),
        scratch_types=[
            pltpu.VMEM((nwin, window), jnp.int32),
            pltpu.VMEM((nwin, window), jnp.int32),
            pltpu.VMEM((ept,), jnp.float32),
            pltpu.VMEM((window, hdim), jnp.float32),
            pltpu.VMEM((window, hdim), jnp.float32),
            pltpu.VMEM_SHARED((n_pad, hdim), jnp.float32),
            pltpu.SemaphoreType.DMA,
            pltpu.SemaphoreType.DMA,
            pltpu.SemaphoreType.DMA,
            pltpu.SemaphoreType.DMA,
        ],
    )
    def k(h2_hbm, src_hbm, dst_hbm, w_hbm, out_hbm,
          src_v, dst_v, w_v, rows_a, rows_b, acc, gsa, gsb, ssa, ssb):
        cid = lax.axis_index("c")
        sid = lax.axis_index("s")
        wid = sid * _NC + cid

        pltpu.sync_copy(src_hbm.at[wid], src_v)
        pltpu.sync_copy(dst_hbm.at[wid], dst_v)
        pltpu.sync_copy(w_hbm.at[wid], w_v)

        zv = jnp.zeros((_LANES,), jnp.float32)

        @pl.loop(0, window)
        def _zfill(i):
            for c in range(hdim // _LANES):
                rows_a[i, pl.ds(c * _LANES, _LANES)] = zv

        @pl.loop(0, rps // window)
        def _zcopy(j):
            pltpu.sync_copy(
                rows_a, acc.at[pl.ds(sid * rps + j * window, window)])

        def g_issue(g, buf, sem):
            pltpu.async_copy(h2_hbm.at[src_v.at[g]], buf, sem)

        def g_wait(g, buf, sem):
            pltpu.make_async_copy(h2_hbm.at[src_v.at[g]], buf, sem).wait()

        def s_issue(g, buf, sem):
            pltpu.async_copy(buf, acc.at[dst_v.at[g]], sem, add=True)

        def s_wait(g, buf, sem):
            pltpu.make_async_copy(buf, acc.at[dst_v.at[g]], sem).wait()

        def scale(g, buf):
            @pl.loop(0, window)
            def _row(i):
                wv = plsc.load_gather(
                    w_v, [jnp.full((_LANES,), g * window + i, jnp.int32)])
                for c in range(hdim // _LANES):
                    sl = (i, pl.ds(c * _LANES, _LANES))
                    buf[sl] = buf[sl] * wv

        g_issue(0, rows_a, gsa)
        if nwin > 1:
            g_issue(1, rows_b, gsb)
        plsc.subcore_barrier()

        npair = (nwin // 2) * 2

        @pl.loop(0, npair, step=2)
        def _pair(g):
            g_wait(g, rows_a, gsa)
            scale(g, rows_a)
            s_issue(g, rows_a, ssa)
            g_wait(g + 1, rows_b, gsb)
            scale(g + 1, rows_b)
            s_issue(g + 1, rows_b, ssb)
            s_wait(g, rows_a, ssa)

            @pl.when(g + 2 < nwin)
            def _():
                g_issue(g + 2, rows_a, gsa)

            s_wait(g + 1, rows_b, ssb)

            @pl.when(g + 3 < nwin)
            def _():
                g_issue(g + 3, rows_b, gsb)

        if nwin % 2:
            g_tail = nwin - 1
            g_wait(g_tail, rows_a, gsa)
            scale(g_tail, rows_a)
            s_issue(g_tail, rows_a, ssa)
            s_wait(g_tail, rows_a, ssa)

        plsc.subcore_barrier()
        pltpu.sync_copy(acc.at[pl.ds(sid * rps, rps)],
                        out_hbm.at[cid, pl.ds(sid * rps, rps)])

    return k(h2, src3, dst3, w2d)


def kernel(node_repesentations, edges, edge_weights,
           Wp1, bp1, Wp2, bp2, Wu1, bu1, Wu2, bu2):
    x = node_repesentations
    n, d = x.shape
    hdim = Wp1.shape[1]

    block_n = 1000 if n % 1000 == 0 else 8
    h2 = _prepare_ffn(x, Wp1, bp1.reshape(1, hdim), Wp2, bp2.reshape(1, hdim),
                      block_n)

    ept = edges.shape[1] // (_NC * _NS)
    window = 80 if ept % 80 == 0 else _LANES
    partials = _sc_spmm(h2, edges[1], edges[0], edge_weights, n, window)

    out = _update_ffn(x, partials,
                      Wu1[:d], Wu1[d:], bu1.reshape(1, hdim),
                      Wu2, bu2.reshape(1, hdim), block_n)
    return out


# speedup vs baseline: 11.6222x; 1.1676x over previous
import dataclasses
import functools

import numpy as np

import jax
import jax.numpy as jnp
from jax import lax
from jax.experimental import pallas as pl
from jax.experimental.pallas import tpu as pltpu
from jax.experimental.pallas import tpu_sc as plsc

_NC = 2
_NS = 16
_LANES = 16


def _gelu(x):
    return 0.5 * x * (1.0 + lax.erf(x * 0.7071067811865476))


def _prepare_body(x_ref, w1_ref, b1_ref, w2_ref, b2_ref, o_ref):
    h = _gelu(
        jnp.dot(x_ref[...], w1_ref[...], preferred_element_type=jnp.float32)
        + b1_ref[...])
    o_ref[...] = _gelu(
        jnp.dot(h, w2_ref[...], preferred_element_type=jnp.float32) + b2_ref[...])


def _prepare_ffn(x, w1, b1, w2, b2, block_n):
    n, d = x.shape
    h = w1.shape[1]
    grid = n // block_n
    return pl.pallas_call(
        _prepare_body,
        grid=(grid,),
        in_specs=[
            pl.BlockSpec((block_n, d), lambda i: (i, 0)),
            pl.BlockSpec((d, h), lambda i: (0, 0)),
            pl.BlockSpec((1, h), lambda i: (0, 0)),
            pl.BlockSpec((h, h), lambda i: (0, 0)),
            pl.BlockSpec((1, h), lambda i: (0, 0)),
        ],
        out_specs=pl.BlockSpec((block_n, h), lambda i: (i, 0)),
        out_shape=jax.ShapeDtypeStruct((n, h), jnp.float32),
    )(x, w1, b1, w2, b2)


def _update_body(x_ref, p0_ref, p1_ref, w1x_ref, w1a_ref, b1_ref, w2_ref,
                 b2_ref, o_ref):
    agg = p0_ref[0] + p1_ref[0]
    h = _gelu(
        jnp.dot(x_ref[...], w1x_ref[...], preferred_element_type=jnp.float32)
        + jnp.dot(agg, w1a_ref[...], preferred_element_type=jnp.float32)
        + b1_ref[...])
    o = _gelu(
        jnp.dot(h, w2_ref[...], preferred_element_type=jnp.float32) + b2_ref[...])
    sq = jnp.sum(o * o, axis=-1, keepdims=True)
    o_ref[...] = o * lax.rsqrt(jnp.maximum(sq, 1e-12))


def _update_ffn(x, partials, w1x, w1a, b1, w2, b2, block_n):
    n, d = x.shape
    h = w2.shape[0]
    grid = n // block_n
    return pl.pallas_call(
        _update_body,
        grid=(grid,),
        in_specs=[
            pl.BlockSpec((block_n, d), lambda i: (i, 0)),
            pl.BlockSpec((1, block_n, h), lambda i: (0, i, 0)),
            pl.BlockSpec((1, block_n, h), lambda i: (1, i, 0)),
            pl.BlockSpec((d, h), lambda i: (0, 0)),
            pl.BlockSpec((h, h), lambda i: (0, 0)),
            pl.BlockSpec((1, h), lambda i: (0, 0)),
            pl.BlockSpec((h, h), lambda i: (0, 0)),
            pl.BlockSpec((1, h), lambda i: (0, 0)),
        ],
        out_specs=pl.BlockSpec((block_n, h), lambda i: (i, 0)),
        out_shape=jax.ShapeDtypeStruct((n, h), jnp.float32),
    )(x, partials, partials, w1x, w1a, b1, w2, b2)


def _sc_spmm(h2, edges, w, n_nodes, window):
    e = edges.shape[1]
    hdim = h2.shape[1]
    nw = _NC * _NS
    ept = e // nw
    nwin = ept // window
    rps = -(-n_nodes // _NS)
    rps = -(-rps // 8) * 8
    n_pad = rps * _NS

    src3 = edges[1].reshape(nw, nwin, window)
    dst3 = edges[0].reshape(nw, nwin, window)
    w2d = w.reshape(nw, ept)
    h2p = jnp.pad(h2, ((0, n_pad - h2.shape[0]), (0, 0)))

    mesh = plsc.VectorSubcoreMesh(core_axis_name="c", subcore_axis_name="s")
    cp = pltpu.CompilerParams()
    for fld, val in (("needs_layout_passes", False),
                     ("use_tc_tiling_on_sc", False)):
        if fld in pltpu.CompilerParams.__dataclass_fields__:
            cp = dataclasses.replace(cp, **{fld: val})

    @functools.partial(
        pl.kernel,
        mesh=mesh,
        compiler_params=cp,
        out_type=jax.ShapeDtypeStruct((_NC, n_pad, hdim), jnp.float32),
        scratch_types=[
            pltpu.VMEM((nwin, window), jnp.int32),
            pltpu.VMEM((nwin, window), jnp.int32),
            pltpu.VMEM((ept,), jnp.float32),
            pltpu.VMEM((window, hdim), jnp.float32),
            pltpu.VMEM((window, hdim), jnp.float32),
            pltpu.VMEM_SHARED((n_pad, hdim), jnp.float32),
            pltpu.VMEM_SHARED((n_pad, hdim), jnp.float32),
            pltpu.SemaphoreType.DMA,
            pltpu.SemaphoreType.DMA,
            pltpu.SemaphoreType.DMA,
            pltpu.SemaphoreType.DMA,
        ],
    )
    def k(h2_hbm, src_hbm, dst_hbm, w_hbm, out_hbm,
          src_v, dst_v, w_v, rows_a, rows_b, acc, h2_s, gsa, gsb, ssa, ssb):
        cid = lax.axis_index("c")
        sid = lax.axis_index("s")
        wid = sid * _NC + cid

        pltpu.sync_copy(src_hbm.at[wid], src_v)
        pltpu.sync_copy(dst_hbm.at[wid], dst_v)
        pltpu.sync_copy(w_hbm.at[wid], w_v)
        pltpu.sync_copy(h2_hbm.at[pl.ds(sid * rps, rps)],
                        h2_s.at[pl.ds(sid * rps, rps)])

        zv = jnp.zeros((_LANES,), jnp.float32)

        @pl.loop(0, window)
        def _zfill(i):
            for c in range(hdim // _LANES):
                rows_a[i, pl.ds(c * _LANES, _LANES)] = zv

        @pl.loop(0, rps // window)
        def _zcopy(j):
            pltpu.sync_copy(
                rows_a, acc.at[pl.ds(sid * rps + j * window, window)])

        def g_issue(g, buf, sem):
            pltpu.async_copy(h2_s.at[src_v.at[g]], buf, sem)

        def g_wait(g, buf, sem):
            pltpu.make_async_copy(h2_s.at[src_v.at[g]], buf, sem).wait()

        def s_issue(g, buf, sem):
            pltpu.async_copy(buf, acc.at[dst_v.at[g]], sem, add=True)

        def s_wait(g, buf, sem):
            pltpu.make_async_copy(buf, acc.at[dst_v.at[g]], sem).wait()

        def scale(g, buf):
            @pl.loop(0, window, step=_LANES)
            def _blk(j):
                rowbase = g * window + j
                for k in range(_LANES):
                    wv = plsc.load_gather(
                        w_v, [jnp.full((_LANES,), rowbase + k, jnp.int32)])
                    for c in range(hdim // _LANES):
                        sl = (j + k, pl.ds(c * _LANES, _LANES))
                        buf[sl] = buf[sl] * wv

        plsc.subcore_barrier()
        g_issue(0, rows_a, gsa)
        if nwin > 1:
            g_issue(1, rows_b, gsb)

        npair = (nwin // 2) * 2

        @pl.loop(0, npair, step=2)
        def _pair(g):
            g_wait(g, rows_a, gsa)
            scale(g, rows_a)
            s_issue(g, rows_a, ssa)
            g_wait(g + 1, rows_b, gsb)
            scale(g + 1, rows_b)
            s_issue(g + 1, rows_b, ssb)
            s_wait(g, rows_a, ssa)

            @pl.when(g + 2 < nwin)
            def _():
                g_issue(g + 2, rows_a, gsa)

            s_wait(g + 1, rows_b, ssb)

            @pl.when(g + 3 < nwin)
            def _():
                g_issue(g + 3, rows_b, gsb)

        if nwin % 2:
            g_tail = nwin - 1
            g_wait(g_tail, rows_a, gsa)
            scale(g_tail, rows_a)
            s_issue(g_tail, rows_a, ssa)
            s_wait(g_tail, rows_a, ssa)

        plsc.subcore_barrier()
        pltpu.sync_copy(acc.at[pl.ds(sid * rps, rps)],
                        out_hbm.at[cid, pl.ds(sid * rps, rps)])

    return k(h2p, src3, dst3, w2d)


def kernel(node_repesentations, edges, edge_weights,
           Wp1, bp1, Wp2, bp2, Wu1, bu1, Wu2, bu2):
    x = node_repesentations
    n, d = x.shape
    hdim = Wp1.shape[1]

    block_n = 1000 if n % 1000 == 0 else 8
    h2 = _prepare_ffn(x, Wp1, bp1.reshape(1, hdim), Wp2, bp2.reshape(1, hdim),
                      block_n)

    ept = edges.shape[1] // (_NC * _NS)
    window = 80 if ept % 80 == 0 else _LANES
    partials = _sc_spmm(h2, edges, edge_weights, n, window)

    out = _update_ffn(x, partials,
                      Wu1[:d], Wu1[d:], bu1.reshape(1, hdim),
                      Wu2, bu2.reshape(1, hdim), block_n)
    return out
